# baseline (speedup 1.0000x reference)
import functools

import jax
import jax.numpy as jnp
from jax import lax
from jax.experimental import pallas as pl
from jax.experimental.pallas import tpu as pltpu
from jax.experimental.pallas import tpu_sc as plsc

N = 4096
D = 2048
DFF = 3072
E = 8
BM = 256
PMAX = N * 2 + E * BM
NBLK = PMAX // BM
NW = 32
CH = 16
ALPHA = 0.02


def _gelu_tanh(v):
    return 0.5 * v * (1.0 + jnp.tanh(jnp.sqrt(2.0 / jnp.pi) * (v + 0.044715 * v ** 3)))


def _routing_body(x_ref, wr_ref, pos_ref, gate_ref, ebm_ref, aux_ref):
    x = x_ref[...]
    wr = wr_ref[...]
    logits = lax.dot_general(x, wr, (((1,), (1,)), ((), ())),
                             preferred_element_type=jnp.float32)

    lane = lax.broadcasted_iota(jnp.int32, (N, E), 1).astype(jnp.float32)
    m1 = jnp.max(logits, axis=1, keepdims=True)
    i1 = jnp.min(jnp.where(logits == m1, lane, jnp.float32(E)), axis=1,
                 keepdims=True)
    oh0 = lane == i1
    masked = jnp.where(oh0, -jnp.inf, logits)
    m2 = jnp.max(masked, axis=1, keepdims=True)
    i2 = jnp.min(jnp.where(masked == m2, lane, jnp.float32(E)), axis=1,
                 keepdims=True)
    oh1 = lane == i2

    e2 = jnp.exp(m2 - m1)
    den = 1.0 + e2
    g1 = 1.0 / den
    g2 = e2 / den

    oh0f = oh0.astype(jnp.float32)
    oh1f = oh1.astype(jnp.float32)
    cnt0 = jnp.sum(oh0f, axis=0, keepdims=True)
    cnt1 = jnp.sum(oh1f, axis=0, keepdims=True)
    cnt_all = cnt0 + cnt1

    both = jnp.concatenate([oh0f, oh1f], axis=1)
    c = both
    sh = 1
    while sh < N:
        c = c + jnp.concatenate(
            [jnp.zeros((sh, 2 * E), jnp.float32), c[: N - sh, :]], axis=0)
        sh *= 2
    excl = c - both
    r0 = excl[:, :E]
    r1 = excl[:, E:]
    rank0 = jnp.sum(oh0f * r0, axis=1, keepdims=True)
    rank1 = jnp.sum(oh1f * r1, axis=1, keepdims=True)

    nb = (cnt_all.astype(jnp.int32) + (BM - 1)) // BM
    nbf = nb.astype(jnp.float32)
    ob = nbf
    s = 1
    while s < E:
        ob = ob + jnp.concatenate(
            [jnp.zeros((1, s), jnp.float32), ob[:, : E - s]], axis=1)
        s *= 2
    off_blocks = ob - nbf
    off = off_blocks * float(BM)

    pos0 = jnp.sum(oh0f * off, axis=1, keepdims=True) + rank0
    pos1 = jnp.sum(oh1f * (off + cnt0), axis=1, keepdims=True) + rank1
    pos_ref[...] = jnp.concatenate([pos0, pos1], axis=1).astype(jnp.int32)
    gate_ref[...] = jnp.concatenate([g1, g2], axis=1)

    jidx = lax.broadcasted_iota(jnp.int32, (128, E), 0).astype(jnp.float32)
    cmp = (jidx >= off_blocks).astype(jnp.float32)
    ebm_ref[...] = (jnp.sum(cmp, axis=1, keepdims=True) - 1.0).astype(jnp.int32)

    gpos0 = (g1 > 0).astype(jnp.float32)
    gpos1 = (g2 > 0).astype(jnp.float32)
    cnt_aux = (jnp.sum(oh0f * gpos0, axis=0, keepdims=True)
               + jnp.sum(oh1f * gpos1, axis=0, keepdims=True))
    gsum = jnp.sum(oh0f * g1 + oh1f * g2, axis=0, keepdims=True)
    f_i = cnt_aux / jnp.sum(cnt_aux)
    m_i = gsum / jnp.maximum(cnt_aux, 1.0)
    aux_ref[...] = jnp.reshape(ALPHA * (jnp.sum(f_i * m_i) / float(E)), (1, 1))


def _routing(x_flat, Wr):
    return pl.pallas_call(
        _routing_body,
        out_shape=[
            jax.ShapeDtypeStruct((N, 2), jnp.int32),
            jax.ShapeDtypeStruct((N, 2), jnp.float32),
            jax.ShapeDtypeStruct((128, 1), jnp.int32),
            jax.ShapeDtypeStruct((1, 1), jnp.float32),
        ],
        compiler_params=pltpu.CompilerParams(
            vmem_limit_bytes=100 * 1024 * 1024),
    )(x_flat, Wr)


@functools.cache
def _sc_mesh():
    return plsc.VectorSubcoreMesh(core_axis_name="c", subcore_axis_name="s")


def _cast_body(w_ref, o_ref):
    o_ref[...] = w_ref[...].astype(jnp.bfloat16)


def _cast_w(W, rows):
    e, a, b = W.shape
    return pl.pallas_call(
        _cast_body,
        grid=(e, a // rows),
        in_specs=[pl.BlockSpec((1, rows, b), lambda i, j: (i, j, 0))],
        out_specs=pl.BlockSpec((1, rows, b), lambda i, j: (i, j, 0)),
        out_shape=jax.ShapeDtypeStruct((e, a, b), jnp.bfloat16),
        compiler_params=pltpu.CompilerParams(
            vmem_limit_bytes=100 * 1024 * 1024),
    )(W)


def _scatter(x_flat, pos0, pos1):
    @functools.partial(
        pl.kernel,
        mesh=_sc_mesh(),
        out_type=jax.ShapeDtypeStruct((PMAX, D), jnp.float32),
        scratch_types=[
            pltpu.VMEM((CH, D), jnp.float32),
            pltpu.VMEM((CH,), jnp.int32),
            pltpu.VMEM((CH,), jnp.int32),
            pltpu.SemaphoreType.DMA,
        ],
    )
    def k(x_hbm, p0_hbm, p1_hbm, xg_hbm, rows_v, i0_v, i1_v, sem):
        wid = lax.axis_index("s") * 2 + lax.axis_index("c")
        base = wid * (N // NW)

        @pl.loop(0, (N // NW) // CH)
        def _(ci):
            b = base + ci * CH
            pltpu.sync_copy(x_hbm.at[pl.ds(b, CH)], rows_v)
            pltpu.sync_copy(p0_hbm.at[pl.ds(b, CH)], i0_v)
            pltpu.sync_copy(p1_hbm.at[pl.ds(b, CH)], i1_v)
            pltpu.async_copy(rows_v, xg_hbm.at[i0_v], sem).wait()
            pltpu.async_copy(rows_v, xg_hbm.at[i1_v], sem).wait()

    return k(x_flat, pos0, pos1)


def _l1_body(ebm_ref, xg_ref, w1_ref, b1_ref, h_ref):
    del ebm_ref
    xb = xg_ref[...].astype(jnp.bfloat16)
    acc = lax.dot_general(xb, w1_ref[0], (((1,), (1,)), ((), ())),
                          preferred_element_type=jnp.float32)
    h_ref[...] = _gelu_tanh(acc + b1_ref[0]).astype(jnp.bfloat16)


def _l1(ebm, xg, W1, b1r):
    return pl.pallas_call(
        _l1_body,
        grid_spec=pltpu.PrefetchScalarGridSpec(
            num_scalar_prefetch=1,
            grid=(NBLK,),
            in_specs=[
                pl.BlockSpec((BM, D), lambda i, ebm: (i, 0)),
                pl.BlockSpec((1, DFF, D), lambda i, ebm: (ebm[i], 0, 0)),
                pl.BlockSpec((1, 1, DFF), lambda i, ebm: (ebm[i], 0, 0)),
            ],
            out_specs=pl.BlockSpec((BM, DFF), lambda i, ebm: (i, 0)),
        ),
        out_shape=jax.ShapeDtypeStruct((PMAX, DFF), jnp.bfloat16),
        compiler_params=pltpu.CompilerParams(
            vmem_limit_bytes=100 * 1024 * 1024),
    )(ebm, xg, W1, b1r)


def _l2_body(ebm_ref, h_ref, w2_ref, b2_ref, og_ref):
    del ebm_ref
    acc = lax.dot_general(h_ref[...], w2_ref[0], (((1,), (1,)), ((), ())),
                          preferred_element_type=jnp.float32)
    og_ref[...] = acc + b2_ref[0]


def _l2(ebm, h, W2, b2r):
    return pl.pallas_call(
        _l2_body,
        grid_spec=pltpu.PrefetchScalarGridSpec(
            num_scalar_prefetch=1,
            grid=(NBLK,),
            in_specs=[
                pl.BlockSpec((BM, DFF), lambda i, ebm: (i, 0)),
                pl.BlockSpec((1, D, DFF), lambda i, ebm: (ebm[i], 0, 0)),
                pl.BlockSpec((1, 1, D), lambda i, ebm: (ebm[i], 0, 0)),
            ],
            out_specs=pl.BlockSpec((BM, D), lambda i, ebm: (i, 0)),
        ),
        out_shape=jax.ShapeDtypeStruct((PMAX, D), jnp.float32),
        compiler_params=pltpu.CompilerParams(
            vmem_limit_bytes=100 * 1024 * 1024),
    )(ebm, h, W2, b2r)


def _gather(og, pos0, pos1):
    @functools.partial(
        pl.kernel,
        mesh=_sc_mesh(),
        out_type=[
            jax.ShapeDtypeStruct((N, D), jnp.float32),
            jax.ShapeDtypeStruct((N, D), jnp.float32),
        ],
        scratch_types=[
            pltpu.VMEM((CH, D), jnp.float32),
            pltpu.VMEM((CH, D), jnp.float32),
            pltpu.VMEM((CH,), jnp.int32),
            pltpu.VMEM((CH,), jnp.int32),
            pltpu.SemaphoreType.DMA,
        ],
    )
    def k(og_hbm, p0_hbm, p1_hbm, g0_hbm, g1_hbm, r0_v, r1_v, i0_v, i1_v, sem):
        wid = lax.axis_index("s") * 2 + lax.axis_index("c")
        base = wid * (N // NW)

        @pl.loop(0, (N // NW) // CH)
        def _(ci):
            b = base + ci * CH
            pltpu.sync_copy(p0_hbm.at[pl.ds(b, CH)], i0_v)
            pltpu.sync_copy(p1_hbm.at[pl.ds(b, CH)], i1_v)
            pltpu.async_copy(og_hbm.at[i0_v], r0_v, sem).wait()
            pltpu.async_copy(og_hbm.at[i1_v], r1_v, sem).wait()
            pltpu.sync_copy(r0_v, g0_hbm.at[pl.ds(b, CH)])
            pltpu.sync_copy(r1_v, g1_hbm.at[pl.ds(b, CH)])

    return k(og, pos0, pos1)


def _combine_body(g0_ref, g1_ref, gt_ref, o_ref):
    o_ref[...] = (g0_ref[...] * gt_ref[:, 0:1] + g1_ref[...] * gt_ref[:, 1:2])


def _combine(g0, g1, gates):
    CB = 512
    return pl.pallas_call(
        _combine_body,
        grid=(N // CB,),
        in_specs=[
            pl.BlockSpec((CB, D), lambda i: (i, 0)),
            pl.BlockSpec((CB, D), lambda i: (i, 0)),
            pl.BlockSpec((CB, 2), lambda i: (i, 0)),
        ],
        out_specs=pl.BlockSpec((CB, D), lambda i: (i, 0)),
        out_shape=jax.ShapeDtypeStruct((N, D), jnp.float32),
    )(g0, g1, gates)


def kernel(x, Wr, W1, b1, W2, b2):
    Bz, Tz, Dz = x.shape
    x_flat = x.reshape(N, D)
    pos, gates, ebm2, aux = _routing(x_flat, Wr)
    pos0 = pos[:, 0]
    pos1 = pos[:, 1]
    ebm = ebm2.reshape(128)
    W1b = _cast_w(W1, 1536)
    W2b = _cast_w(W2, 1024)
    xg = _scatter(x_flat, pos0, pos1)
    h = _l1(ebm, xg, W1b, b1.reshape(E, 1, DFF))
    og = _l2(ebm, h, W2b, b2.reshape(E, 1, D))
    g0, g1 = _gather(og, pos0, pos1)
    out = _combine(g0, g1, gates)
    return out.reshape(Bz, Tz, Dz), aux.reshape(())


# speedup vs baseline: 1.2472x; 1.2472x over previous
import functools

import jax
import jax.numpy as jnp
from jax import lax
from jax.experimental import pallas as pl
from jax.experimental.pallas import tpu as pltpu
from jax.experimental.pallas import tpu_sc as plsc

N = 4096
D = 2048
DFF = 3072
E = 8
BM = 256
PMAX = N * 2 + E * BM
NBLK = PMAX // BM
NW = 32
TPW = N // NW
SCH = 16
GCH = 8
ALPHA = 0.02


def _gelu_tanh(v):
    return 0.5 * v * (1.0 + jnp.tanh(jnp.sqrt(2.0 / jnp.pi) * (v + 0.044715 * v ** 3)))


def _routing_body(x_ref, wr_ref, pos_ref, gate_ref, ebm_ref, nact_ref, aux_ref):
    x = x_ref[...]
    wr = wr_ref[...]
    logits = lax.dot_general(x, wr, (((1,), (1,)), ((), ())),
                             preferred_element_type=jnp.float32)

    lane = lax.broadcasted_iota(jnp.int32, (N, E), 1).astype(jnp.float32)
    m1 = jnp.max(logits, axis=1, keepdims=True)
    i1 = jnp.min(jnp.where(logits == m1, lane, jnp.float32(E)), axis=1,
                 keepdims=True)
    oh0 = lane == i1
    masked = jnp.where(oh0, -jnp.inf, logits)
    m2 = jnp.max(masked, axis=1, keepdims=True)
    i2 = jnp.min(jnp.where(masked == m2, lane, jnp.float32(E)), axis=1,
                 keepdims=True)
    oh1 = lane == i2

    e2 = jnp.exp(m2 - m1)
    den = 1.0 + e2
    g1 = 1.0 / den
    g2 = e2 / den

    oh0f = oh0.astype(jnp.float32)
    oh1f = oh1.astype(jnp.float32)
    cnt0 = jnp.sum(oh0f, axis=0, keepdims=True)
    cnt1 = jnp.sum(oh1f, axis=0, keepdims=True)
    cnt_all = cnt0 + cnt1

    both = jnp.concatenate([oh0f, oh1f], axis=1)
    c = both
    sh = 1
    while sh < N:
        c = c + jnp.concatenate(
            [jnp.zeros((sh, 2 * E), jnp.float32), c[: N - sh, :]], axis=0)
        sh *= 2
    excl = c - both
    r0 = excl[:, :E]
    r1 = excl[:, E:]
    rank0 = jnp.sum(oh0f * r0, axis=1, keepdims=True)
    rank1 = jnp.sum(oh1f * r1, axis=1, keepdims=True)

    nb = (cnt_all.astype(jnp.int32) + (BM - 1)) // BM
    nbf = nb.astype(jnp.float32)
    ob = nbf
    s = 1
    while s < E:
        ob = ob + jnp.concatenate(
            [jnp.zeros((1, s), jnp.float32), ob[:, : E - s]], axis=1)
        s *= 2
    off_blocks = ob - nbf
    off = off_blocks * float(BM)

    pos0 = jnp.sum(oh0f * off, axis=1, keepdims=True) + rank0
    pos1 = jnp.sum(oh1f * (off + cnt0), axis=1, keepdims=True) + rank1
    pos_ref[...] = jnp.concatenate([pos0, pos1], axis=1).astype(jnp.int32)
    gate_ref[...] = jnp.concatenate([g1, g2], axis=1)

    jidx = lax.broadcasted_iota(jnp.int32, (128, E), 0).astype(jnp.float32)
    cmp = (jidx >= off_blocks).astype(jnp.float32)
    ebm_ref[...] = (jnp.sum(cmp, axis=1, keepdims=True) - 1.0).astype(jnp.int32)
    nact_ref[...] = jnp.reshape(jnp.sum(nbf), (1, 1)).astype(jnp.int32)

    gpos0 = (g1 > 0).astype(jnp.float32)
    gpos1 = (g2 > 0).astype(jnp.float32)
    cnt_aux = (jnp.sum(oh0f * gpos0, axis=0, keepdims=True)
               + jnp.sum(oh1f * gpos1, axis=0, keepdims=True))
    gsum = jnp.sum(oh0f * g1 + oh1f * g2, axis=0, keepdims=True)
    f_i = cnt_aux / jnp.sum(cnt_aux)
    m_i = gsum / jnp.maximum(cnt_aux, 1.0)
    aux_ref[...] = jnp.reshape(ALPHA * (jnp.sum(f_i * m_i) / float(E)), (1, 1))


def _routing(x_flat, Wr):
    return pl.pallas_call(
        _routing_body,
        out_shape=[
            jax.ShapeDtypeStruct((N, 2), jnp.int32),
            jax.ShapeDtypeStruct((N, 2), jnp.float32),
            jax.ShapeDtypeStruct((128, 1), jnp.int32),
            jax.ShapeDtypeStruct((1, 1), jnp.int32),
            jax.ShapeDtypeStruct((1, 1), jnp.float32),
        ],
        compiler_params=pltpu.CompilerParams(
            vmem_limit_bytes=100 * 1024 * 1024),
    )(x_flat, Wr)


@functools.cache
def _sc_mesh():
    return plsc.VectorSubcoreMesh(core_axis_name="c", subcore_axis_name="s")


def _scatter(x_flat, p0r, p1r):
    nch = TPW // SCH

    @functools.partial(
        pl.kernel,
        mesh=_sc_mesh(),
        out_type=jax.ShapeDtypeStruct((PMAX, D), jnp.float32),
        scratch_types=[
            pltpu.VMEM((SCH, D), jnp.float32),
            pltpu.VMEM((SCH, D), jnp.float32),
            pltpu.VMEM((nch, SCH), jnp.int32),
            pltpu.VMEM((nch, SCH), jnp.int32),
            pltpu.SemaphoreType.DMA,
            pltpu.SemaphoreType.DMA,
            pltpu.SemaphoreType.DMA,
            pltpu.SemaphoreType.DMA,
        ],
    )
    def k(x_hbm, p0_hbm, p1_hbm, xg_hbm, rA, rB, i0, i1, lsA, lsB, ssA, ssB):
        wid = lax.axis_index("s") * 2 + lax.axis_index("c")
        base = wid * TPW
        pltpu.sync_copy(p0_hbm.at[wid], i0)
        pltpu.sync_copy(p1_hbm.at[wid], i1)

        @pl.loop(0, nch, step=2)
        def _(c):
            ldA = pltpu.async_copy(
                x_hbm.at[pl.ds(base + c * SCH, SCH)], rA, lsA)
            ldB = pltpu.async_copy(
                x_hbm.at[pl.ds(base + (c + 1) * SCH, SCH)], rB, lsB)
            ldA.wait()
            sA0 = pltpu.async_copy(rA, xg_hbm.at[i0.at[c]], ssA)
            sA1 = pltpu.async_copy(rA, xg_hbm.at[i1.at[c]], ssA)
            ldB.wait()
            sB0 = pltpu.async_copy(rB, xg_hbm.at[i0.at[c + 1]], ssB)
            sB1 = pltpu.async_copy(rB, xg_hbm.at[i1.at[c + 1]], ssB)
            sA0.wait()
            sA1.wait()
            sB0.wait()
            sB1.wait()

    return k(x_flat, p0r, p1r)


def _l1_body(ebm_ref, nact_ref, xg_ref, w1_ref, b1_ref, h_ref):
    del ebm_ref

    @pl.when(pl.program_id(0) < nact_ref[0])
    def _():
        acc = lax.dot_general(xg_ref[...], w1_ref[0], (((1,), (1,)), ((), ())),
                              preferred_element_type=jnp.float32)
        h_ref[...] = _gelu_tanh(acc + b1_ref[0])


def _l1(ebm, nact, xg, W1, b1r):
    return pl.pallas_call(
        _l1_body,
        grid_spec=pltpu.PrefetchScalarGridSpec(
            num_scalar_prefetch=2,
            grid=(NBLK,),
            in_specs=[
                pl.BlockSpec((BM, D), lambda i, ebm, nact: (i, 0)),
                pl.BlockSpec((1, DFF, D), lambda i, ebm, nact: (ebm[i], 0, 0)),
                pl.BlockSpec((1, 1, DFF), lambda i, ebm, nact: (ebm[i], 0, 0)),
            ],
            out_specs=pl.BlockSpec((BM, DFF), lambda i, ebm, nact: (i, 0)),
        ),
        out_shape=jax.ShapeDtypeStruct((PMAX, DFF), jnp.float32),
        compiler_params=pltpu.CompilerParams(
            vmem_limit_bytes=100 * 1024 * 1024),
    )(ebm, nact, xg, W1, b1r)


def _l2_body(ebm_ref, nact_ref, h_ref, w2_ref, b2_ref, og_ref):
    del ebm_ref

    @pl.when(pl.program_id(0) < nact_ref[0])
    def _():
        acc = lax.dot_general(h_ref[...], w2_ref[0], (((1,), (1,)), ((), ())),
                              preferred_element_type=jnp.float32)
        og_ref[...] = acc + b2_ref[0]


def _l2(ebm, nact, h, W2, b2r):
    return pl.pallas_call(
        _l2_body,
        grid_spec=pltpu.PrefetchScalarGridSpec(
            num_scalar_prefetch=2,
            grid=(NBLK,),
            in_specs=[
                pl.BlockSpec((BM, DFF), lambda i, ebm, nact: (i, 0)),
                pl.BlockSpec((1, D, DFF), lambda i, ebm, nact: (ebm[i], 0, 0)),
                pl.BlockSpec((1, 1, D), lambda i, ebm, nact: (ebm[i], 0, 0)),
            ],
            out_specs=pl.BlockSpec((BM, D), lambda i, ebm, nact: (i, 0)),
        ),
        out_shape=jax.ShapeDtypeStruct((PMAX, D), jnp.float32),
        compiler_params=pltpu.CompilerParams(
            vmem_limit_bytes=100 * 1024 * 1024),
    )(ebm, nact, h, W2, b2r)


def _gather(og, p0r, p1r):
    nch = TPW // GCH

    @functools.partial(
        pl.kernel,
        mesh=_sc_mesh(),
        out_type=[
            jax.ShapeDtypeStruct((N, D), jnp.float32),
            jax.ShapeDtypeStruct((N, D), jnp.float32),
        ],
        scratch_types=[
            pltpu.VMEM((GCH, D), jnp.float32),
            pltpu.VMEM((GCH, D), jnp.float32),
            pltpu.VMEM((GCH, D), jnp.float32),
            pltpu.VMEM((GCH, D), jnp.float32),
            pltpu.VMEM((nch, GCH), jnp.int32),
            pltpu.VMEM((nch, GCH), jnp.int32),
            pltpu.SemaphoreType.DMA,
            pltpu.SemaphoreType.DMA,
            pltpu.SemaphoreType.DMA,
            pltpu.SemaphoreType.DMA,
        ],
    )
    def k(og_hbm, p0_hbm, p1_hbm, g0_hbm, g1_hbm,
          rA0, rA1, rB0, rB1, i0, i1, lsA, lsB, ssA, ssB):
        wid = lax.axis_index("s") * 2 + lax.axis_index("c")
        base = wid * TPW
        pltpu.sync_copy(p0_hbm.at[wid], i0)
        pltpu.sync_copy(p1_hbm.at[wid], i1)

        @pl.loop(0, nch, step=2)
        def _(c):
            gA0 = pltpu.async_copy(og_hbm.at[i0.at[c]], rA0, lsA)
            gA1 = pltpu.async_copy(og_hbm.at[i1.at[c]], rA1, lsA)
            gB0 = pltpu.async_copy(og_hbm.at[i0.at[c + 1]], rB0, lsB)
            gB1 = pltpu.async_copy(og_hbm.at[i1.at[c + 1]], rB1, lsB)
            gA0.wait()
            gA1.wait()
            oA = base + c * GCH
            sA0 = pltpu.async_copy(rA0, g0_hbm.at[pl.ds(oA, GCH)], ssA)
            sA1 = pltpu.async_copy(rA1, g1_hbm.at[pl.ds(oA, GCH)], ssA)
            gB0.wait()
            gB1.wait()
            oB = base + (c + 1) * GCH
            sB0 = pltpu.async_copy(rB0, g0_hbm.at[pl.ds(oB, GCH)], ssB)
            sB1 = pltpu.async_copy(rB1, g1_hbm.at[pl.ds(oB, GCH)], ssB)
            sA0.wait()
            sA1.wait()
            sB0.wait()
            sB1.wait()

    return k(og, p0r, p1r)


def _combine_body(g0_ref, g1_ref, gt_ref, o_ref):
    o_ref[...] = (g0_ref[...] * gt_ref[:, 0:1] + g1_ref[...] * gt_ref[:, 1:2])


def _combine(g0, g1, gates):
    CB = 512
    return pl.pallas_call(
        _combine_body,
        grid=(N // CB,),
        in_specs=[
            pl.BlockSpec((CB, D), lambda i: (i, 0)),
            pl.BlockSpec((CB, D), lambda i: (i, 0)),
            pl.BlockSpec((CB, 2), lambda i: (i, 0)),
        ],
        out_specs=pl.BlockSpec((CB, D), lambda i: (i, 0)),
        out_shape=jax.ShapeDtypeStruct((N, D), jnp.float32),
    )(g0, g1, gates)


def kernel(x, Wr, W1, b1, W2, b2):
    Bz, Tz, Dz = x.shape
    x_flat = x.reshape(N, D)
    pos, gates, ebm2, nact2, aux = _routing(x_flat, Wr)
    pos0 = pos[:, 0]
    pos1 = pos[:, 1]
    ebm = ebm2.reshape(128)
    nact = nact2.reshape(1)
    xg = _scatter(x_flat,
                  pos0.reshape(NW, TPW // SCH, SCH),
                  pos1.reshape(NW, TPW // SCH, SCH))
    h = _l1(ebm, nact, xg, W1, b1.reshape(E, 1, DFF))
    og = _l2(ebm, nact, h, W2, b2.reshape(E, 1, D))
    g0, g1 = _gather(og,
                     pos0.reshape(NW, TPW // GCH, GCH),
                     pos1.reshape(NW, TPW // GCH, GCH))
    out = _combine(g0, g1, gates)
    return out.reshape(Bz, Tz, Dz), aux.reshape(())
